# Initial kernel scaffold; baseline (speedup 1.0000x reference)
import jax
import jax.numpy as jnp
from jax.experimental import pallas as pl


def kernel(img_feat, cloud, cloud_tar, W1, b1, W2, b2, Wps1, bps1, Wps2, bps2, Wp1, bp1, Wp2, bp2, Wf1, bf1, Wf2, bf2, Wf, bf):
    raise NotImplementedError("write your pallas kernel here")


# speedup vs baseline: 4.1326x; 4.1326x over previous
import jax
import jax.numpy as jnp
from jax.experimental import pallas as pl
from jax.experimental.pallas import tpu as pltpu

NP_ = 8
N_ = 500
NPAD = 512
BIG = 1e30


def _lrelu(t):
    return jnp.where(t >= 0, t, 0.01 * t)


def _fused_body(Pr, Pc, Tr, Tc, Cr, Gr,
                W1t, b1, W2t, b2, Wps1t, bps1, Wps2t, bps2,
                Wp1t, bp1, Wp2t, bp2,
                Wf1at, Wf1bt, bf1, Wf2at, Wf2bt, bf2,
                Wfat, Wfbt, bf, out_ref):
    P = Pr[...]
    T = Tr[...]
    C = Cr[...]
    G = Gr[...]
    Pcv = Pc[...]
    Tcv = Tc[...]
    (W1t, b1, W2t, b2, Wps1t, bps1, Wps2t, bps2, Wp1t, bp1, Wp2t, bp2,
     Wf1at, Wf1bt, bf1, Wf2at, Wf2bt, bf2, Wfat, Wfbt, bf) = [
        r[...] for r in (W1t, b1, W2t, b2, Wps1t, bps1, Wps2t, bps2,
                         Wp1t, bp1, Wp2t, bp2, Wf1at, Wf1bt, bf1,
                         Wf2at, Wf2bt, bf2, Wfat, Wfbt, bf)]

    dot = lambda a, b: jnp.dot(a, b, preferred_element_type=jnp.float32)

    pn = jnp.sum(P * P, axis=1, keepdims=True)
    tn = jnp.sum(T * T, axis=1, keepdims=True)
    cn = jnp.sum(C * C, axis=1, keepdims=True)
    tnc = jnp.sum(Tcv * Tcv, axis=0, keepdims=True)
    pnc = jnp.sum(Pcv * Pcv, axis=0, keepdims=True)

    d2 = pn + tnc - 2.0 * dot(P, Tcv)
    d2t = tn + pnc - 2.0 * dot(T, Pcv)
    e1 = cn + tnc - 2.0 * dot(C, Tcv)

    col_iota = jax.lax.broadcasted_iota(jnp.int32, (1, NPAD), 1)
    row_iota = jax.lax.broadcasted_iota(jnp.int32, (NPAD, 1), 0)
    col_pad = col_iota >= N_

    def top8(dd, extract):
        dd = jnp.where(col_pad, BIG, dd)
        idxs, vals = [], []
        for _ in range(NP_):
            rowmin = jnp.min(dd, axis=1, keepdims=True)
            cand = jnp.where(dd == rowmin, col_iota, NPAD)
            mstar = jnp.min(cand, axis=1, keepdims=True)
            mask = col_iota == mstar
            idxs.append(mstar)
            if extract is None:
                vals.append(rowmin)
            else:
                vals.append(jnp.sum(jnp.where(mask, extract, 0.0),
                                    axis=1, keepdims=True))
            dd = jnp.where(mask, BIG, dd)
        return idxs, vals

    idx1, v1 = top8(d2, e1)
    idx2, v2 = top8(d2t, None)

    def weights(vals):
        l = jnp.concatenate([-jnp.sqrt(jnp.maximum(v, 0.0)) for v in vals],
                            axis=1)
        l = jnp.where(row_iota < N_, l, -BIG)
        m = jnp.max(l)
        e = jnp.exp(l - m)
        return e * (1.0 / (NP_ * jnp.sum(e)))

    w1 = weights(v1)
    w2 = weights(v2)

    A1 = jnp.zeros((NPAD, NPAD), jnp.float32)
    A2 = jnp.zeros((NPAD, NPAD), jnp.float32)
    for j in range(NP_):
        A1 = A1 + w1[:, j:j + 1] * (col_iota == idx1[j]).astype(jnp.float32)
        A2 = A2 + w2[:, j:j + 1] * (col_iota == idx2[j]).astype(jnp.float32)

    def mlp2(X, Wat, ba, Wbt, bb):
        return dot(_lrelu(dot(X, Wat) + ba[...]), Wbt) + bb[...]

    cf = mlp2(P, Wp1t, bp1, Wp2t, bp2)
    sfull = mlp2(G, W1t, b1, W2t, b2)
    spfull = mlp2(cf, Wps1t, bps1, Wps2t, bps2)

    sf = dot(A1, sfull)
    sfp = dot(A2, spfull)

    final1 = dot(sf, Wf1at) + dot(cf, Wf1bt) + bf1[...]
    final2 = dot(sfp, Wf2at) + dot(G, Wf2bt) + bf2[...]
    out = dot(_lrelu(final2), Wfat) + dot(_lrelu(final1), Wfbt) + bf[...]
    out_ref[...] = out


def kernel(img_feat, cloud, cloud_tar, W1, b1, W2, b2, Wps1, bps1, Wps2, bps2,
           Wp1, bp1, Wp2, bp2, Wf1, bf1, Wf2, bf2, Wf, bf):
    f32 = jnp.float32

    def padr(x, rows=NPAD):
        return jnp.pad(x, ((0, rows - x.shape[0]), (0, 0)))

    P3 = cloud[0]
    T3 = cloud_tar[0]
    C3 = cloud.reshape(3, N_).T

    Pr = padr(jnp.pad(P3, ((0, 0), (0, 5))))
    Tr = padr(jnp.pad(T3, ((0, 0), (0, 5))))
    Cr = padr(jnp.pad(C3, ((0, 0), (0, 5))))
    Pc = Pr.T[:8]
    Tc = Tr.T[:8]
    Gr = padr(img_feat[0].T)

    row2 = lambda b: b[None, :].astype(f32)
    args = (
        Pr.astype(f32), Pc.astype(f32), Tr.astype(f32), Tc.astype(f32),
        Cr.astype(f32), Gr.astype(f32),
        W1.T.astype(f32), row2(b1), W2.T.astype(f32), row2(b2),
        Wps1.T.astype(f32), row2(bps1), Wps2.T.astype(f32), row2(bps2),
        jnp.pad(Wp1.T, ((0, 5), (0, 0))).astype(f32), row2(bp1),
        Wp2.T.astype(f32), row2(bp2),
        Wf1[:, :128].T.astype(f32), Wf1[:, 128:].T.astype(f32), row2(bf1),
        Wf2[:, :128].T.astype(f32), Wf2[:, 128:].T.astype(f32), row2(bf2),
        Wf[:, :64].T.astype(f32), Wf[:, 64:].T.astype(f32), row2(bf),
    )

    out = pl.pallas_call(
        _fused_body,
        out_shape=jax.ShapeDtypeStruct((NPAD, 64), f32),
        in_specs=[pl.BlockSpec(memory_space=pltpu.VMEM) for _ in args],
        out_specs=pl.BlockSpec(memory_space=pltpu.VMEM),
    )(*args)

    return out[:N_].T[None]
